# Initial kernel scaffold; baseline (speedup 1.0000x reference)
import jax
import jax.numpy as jnp
from jax.experimental import pallas as pl


def kernel(batch_embeddings, position_mask, W, b, gamma, beta):
    raise NotImplementedError("write your pallas kernel here")


# speedup vs baseline: 3.4952x; 3.4952x over previous
import jax
import jax.numpy as jnp
from jax.experimental import pallas as pl
from jax.experimental.pallas import tpu as pltpu

H = 768
BLK = 1024


def _dense_body(x_ref, m_ref, wt_ref, b_ref, g_ref, be_ref, o_ref):
    x = x_ref[...]
    soft = jax.lax.dot_general(
        x.astype(jnp.bfloat16), wt_ref[...],
        (((1,), (0,)), ((), ())),
        preferred_element_type=jnp.float32,
    ) + b_ref[...]
    z = x + soft
    mean = jnp.mean(z, axis=-1, keepdims=True)
    zc = z - mean
    var = jnp.mean(zc * zc, axis=-1, keepdims=True)
    normed = zc * jax.lax.rsqrt(var + 1e-5) * g_ref[...] + be_ref[...]
    sel = m_ref[...] == 1
    o_ref[...] = jnp.where(sel, normed, x)


def kernel(batch_embeddings, position_mask, W, b, gamma, beta):
    B, S, Hh = batch_embeddings.shape
    N = B * S
    x = batch_embeddings.reshape(N, Hh)
    m = position_mask.reshape(N, 1).astype(jnp.int32)
    wt = W.T.astype(jnp.bfloat16)
    b2 = b.reshape(1, Hh)
    g2 = gamma.reshape(1, Hh)
    be2 = beta.reshape(1, Hh)

    out = pl.pallas_call(
        _dense_body,
        grid=(N // BLK,),
        in_specs=[
            pl.BlockSpec((BLK, Hh), lambda i: (i, 0)),
            pl.BlockSpec((BLK, 1), lambda i: (i, 0)),
            pl.BlockSpec((Hh, Hh), lambda i: (0, 0)),
            pl.BlockSpec((1, Hh), lambda i: (0, 0)),
            pl.BlockSpec((1, Hh), lambda i: (0, 0)),
            pl.BlockSpec((1, Hh), lambda i: (0, 0)),
        ],
        out_specs=pl.BlockSpec((BLK, Hh), lambda i: (i, 0)),
        out_shape=jax.ShapeDtypeStruct((N, Hh), jnp.float32),
        compiler_params=pltpu.CompilerParams(
            dimension_semantics=("arbitrary",),
        ),
    )(x, m, wt, b2, g2, be2)
    return out.reshape(B, S, Hh)
